# baseline (speedup 1.0000x reference)
import functools

import jax
import jax.numpy as jnp
import numpy as np
from jax import lax
from jax.experimental import pallas as pl
from jax.experimental.pallas import tpu as pltpu
from jax.experimental.pallas import tpu_sc as plsc

_BS = 1024


def _threefry_bits_u32(x1):
    ks0 = jnp.uint32(0)
    ks1 = jnp.uint32(42)
    ks2 = jnp.uint32(0x1BD11BDA) ^ ks0 ^ ks1
    x0 = jnp.zeros_like(x1) + ks0

    def rounds(x0, x1, rots):
        for d in rots:
            x0 = x0 + x1
            x1 = (x1 << d) | (x1 >> (32 - d))
            x1 = x1 ^ x0
        return x0, x1

    r_a = (13, 15, 26, 6)
    r_b = (17, 29, 16, 24)
    for i, (a0, a1, rots) in enumerate(
        [(ks1, ks2, r_a), (ks2, ks0, r_b), (ks0, ks1, r_a),
         (ks1, ks2, r_b), (ks2, ks0, r_a)]
    ):
        x0, x1 = rounds(x0, x1, rots)
        x0 = x0 + a0
        x1 = x1 + a1 + jnp.uint32(i + 1)
    return x0 ^ x1


def _body(n, bs, num_pairs, bi_ref, bj_ref, probs_ref, out_ref,
          up_buf, tr_buf, iota_buf, sems):
    p = pl.program_id(0)
    slot = lax.rem(p, 2)
    bi = bi_ref[p]
    bj = bj_ref[p]

    @pl.when(p == 0)
    def _init_iota():
        rl = lax.broadcasted_iota(jnp.uint32, (bs, bs), 0)
        cl = lax.broadcasted_iota(jnp.uint32, (bs, bs), 1)
        iota_buf[...] = rl * jnp.uint32(n) + cl

    def copy_desc(buf, row_blk, col_blk, sem_idx):
        return pltpu.make_async_copy(
            buf.at[slot],
            out_ref.at[pl.ds(row_blk * bs, bs), pl.ds(col_blk * bs, bs)],
            sems.at[slot, sem_idx],
        )

    @pl.when(p >= 2)
    def _drain_prev():
        copy_desc(up_buf, bi, bj, 0).wait()
        copy_desc(tr_buf, bj, bi, 1).wait()

    base = (bi * (bs * n) + bj * bs + 42).astype(jnp.uint32)
    bits = _threefry_bits_u32(iota_buf[...] + base)
    fbits = (bits >> 9) | jnp.uint32(0x3F800000)
    noise = lax.bitcast_convert_type(fbits, jnp.float32) - 1.0
    bern = (noise < probs_ref[...]).astype(jnp.float32)
    bern_t = bern.T
    diag = bi == bj

    @pl.when(jnp.logical_not(diag))
    def _off_diag():
        up_buf[slot] = bern
        tr_buf[slot] = bern_t
        copy_desc(tr_buf, bj, bi, 1).start()

    @pl.when(diag)
    def _diag():
        rl = lax.broadcasted_iota(jnp.int32, (bs, bs), 0)
        cl = lax.broadcasted_iota(jnp.int32, (bs, bs), 1)
        up_buf[slot] = jnp.where(rl > cl, bern_t, bern)
        copy_desc(up_buf, bj, bi, 1).start()

    copy_desc(up_buf, bi, bj, 0).start()

    @pl.when(p == num_pairs - 1)
    def _drain_tail():
        copy_desc(up_buf, bi, bj, 0).wait()
        copy_desc(tr_buf, bj, bi, 1).wait()
        if num_pairs >= 2:
            other = 1 - slot
            pltpu.make_async_copy(
                up_buf.at[other],
                out_ref.at[pl.ds(0, bs), pl.ds(0, bs)],
                sems.at[other, 0],
            ).wait()
            pltpu.make_async_copy(
                tr_buf.at[other],
                out_ref.at[pl.ds(0, bs), pl.ds(0, bs)],
                sems.at[other, 1],
            ).wait()


_SC_BLOCKS = [(d, d) for d in range(8)] + [(0, 1), (2, 3), (4, 5), (6, 7)]


def _sc_bits_block(n, bs):
    mesh = plsc.VectorSubcoreMesh(core_axis_name="c", subcore_axis_name="s")
    rows_per_w = bs // 32
    nblk = len(_SC_BLOCKS)

    @functools.partial(
        pl.kernel,
        mesh=mesh,
        out_type=jax.ShapeDtypeStruct((nblk, bs, bs), jnp.uint32),
        scratch_types=[
            pltpu.VMEM((16,), jnp.uint32),
            pltpu.VMEM((2, bs), jnp.uint32),
            pltpu.SemaphoreType.DMA((2,)),
        ],
    )
    def k(lane_hbm, out_hbm, lane_v, row_v, sems):
        wid = lax.axis_index("c") * 16 + lax.axis_index("s")
        pltpu.sync_copy(lane_hbm, lane_v)
        lane = lane_v[...]

        for b, (bi, bj) in enumerate(_SC_BLOCKS):
            base_const = jnp.uint32((bi * bs * n + bj * bs + 42) & 0xFFFFFFFF)

            @pl.loop(0, rows_per_w)
            def _row(i, b=b, base_const=base_const):
                row = wid * rows_per_w + i
                slot = lax.rem(i, 2)
                rbase = row.astype(jnp.uint32) * jnp.uint32(n) + base_const

                @pl.when(i >= 2)
                def _():
                    pltpu.make_async_copy(
                        row_v.at[slot], out_hbm.at[b, row], sems.at[slot]
                    ).wait()

                @pl.loop(0, bs, step=32)
                def _chunk(c, rbase=rbase, slot=slot):
                    x1a = lane + (rbase + c.astype(jnp.uint32))
                    row_v[slot, pl.ds(c, 16)] = _threefry_bits_u32(x1a)
                    row_v[slot, pl.ds(c + 16, 16)] = _threefry_bits_u32(
                        x1a + jnp.uint32(16))

                pltpu.make_async_copy(
                    row_v.at[slot], out_hbm.at[b, row], sems.at[slot]
                ).start()

            pltpu.make_async_copy(
                row_v.at[lax.rem(rows_per_w - 1, 2)],
                out_hbm.at[b, 0], sems.at[lax.rem(rows_per_w - 1, 2)],
            ).wait()
            pltpu.make_async_copy(
                row_v.at[lax.rem(rows_per_w, 2)],
                out_hbm.at[b, 0], sems.at[lax.rem(rows_per_w, 2)],
            ).wait()

    lane_arr = jnp.arange(16, dtype=jnp.uint32)
    return k(lane_arr)


def _stitch_body(bs, num_blk, tc_in_ref, bits_ref, probs_hbm, bi_ref, bj_ref,
                 out_ref, pbuf, up_buf, tr_buf, sems):
    del tc_in_ref
    d = pl.program_id(0)
    slot = lax.rem(d, 2)
    bi = bi_ref[d]
    bj = bj_ref[d]

    def copy_desc(buf, row_blk, col_blk, sem_idx):
        return pltpu.make_async_copy(
            buf.at[slot],
            out_ref.at[pl.ds(row_blk * bs, bs), pl.ds(col_blk * bs, bs)],
            sems.at[slot, sem_idx],
        )

    probs_in = pltpu.make_async_copy(
        probs_hbm.at[pl.ds(bi * bs, bs), pl.ds(bj * bs, bs)],
        pbuf.at[slot],
        sems.at[slot, 2],
    )
    probs_in.start()

    @pl.when(d >= 2)
    def _drain_prev():
        copy_desc(up_buf, bi, bj, 0).wait()
        copy_desc(tr_buf, bj, bi, 1).wait()

    probs_in.wait()

    fbits = (bits_ref[0] >> 9) | jnp.uint32(0x3F800000)
    noise = lax.bitcast_convert_type(fbits, jnp.float32) - 1.0
    bern = (noise < pbuf[slot]).astype(jnp.float32)
    bern_t = bern.T
    diag = bi == bj

    @pl.when(jnp.logical_not(diag))
    def _off_diag():
        up_buf[slot] = bern
        tr_buf[slot] = bern_t
        copy_desc(tr_buf, bj, bi, 1).start()

    @pl.when(diag)
    def _diag():
        rl = lax.broadcasted_iota(jnp.int32, (bs, bs), 0)
        cl = lax.broadcasted_iota(jnp.int32, (bs, bs), 1)
        up_buf[slot] = jnp.where(rl > cl, bern_t, bern)
        copy_desc(up_buf, bj, bi, 1).start()

    copy_desc(up_buf, bi, bj, 0).start()

    @pl.when(d == num_blk - 1)
    def _drain_tail():
        copy_desc(up_buf, bi, bj, 0).wait()
        copy_desc(tr_buf, bj, bi, 1).wait()
        if num_blk >= 2:
            other = 1 - slot
            pltpu.make_async_copy(
                up_buf.at[other],
                out_ref.at[pl.ds(0, bs), pl.ds(0, bs)],
                sems.at[other, 0],
            ).wait()
            pltpu.make_async_copy(
                tr_buf.at[other],
                out_ref.at[pl.ds(0, bs), pl.ds(0, bs)],
                sems.at[other, 1],
            ).wait()


def kernel(edge_probs):
    n = edge_probs.shape[0]
    bs = _BS
    nb = n // bs
    sc_set = set(_SC_BLOCKS)
    pairs = [(i, j) for i in range(nb) for j in range(i, nb)
             if (i, j) not in sc_set]
    bi_arr = jnp.asarray(np.array([ij[0] for ij in pairs], dtype=np.int32))
    bj_arr = jnp.asarray(np.array([ij[1] for ij in pairs], dtype=np.int32))
    num_pairs = len(pairs)

    grid_spec = pltpu.PrefetchScalarGridSpec(
        num_scalar_prefetch=2,
        grid=(num_pairs,),
        in_specs=[
            pl.BlockSpec((bs, bs), lambda p, bi, bj: (bi[p], bj[p])),
        ],
        out_specs=pl.BlockSpec(memory_space=pl.ANY),
        scratch_shapes=[
            pltpu.VMEM((2, bs, bs), jnp.float32),
            pltpu.VMEM((2, bs, bs), jnp.float32),
            pltpu.VMEM((bs, bs), jnp.uint32),
            pltpu.SemaphoreType.DMA((2, 2)),
        ],
    )
    tc_out = pl.pallas_call(
        functools.partial(_body, n, bs, num_pairs),
        grid_spec=grid_spec,
        out_shape=jax.ShapeDtypeStruct((n, n), jnp.float32),
        compiler_params=pltpu.CompilerParams(
            dimension_semantics=("arbitrary",),
        ),
    )(bi_arr, bj_arr, edge_probs)

    sc_bits = _sc_bits_block(n, bs)
    num_blk = len(_SC_BLOCKS)
    bi_sc = jnp.asarray([b[0] for b in _SC_BLOCKS], dtype=jnp.int32)
    bj_sc = jnp.asarray([b[1] for b in _SC_BLOCKS], dtype=jnp.int32)
    return pl.pallas_call(
        functools.partial(_stitch_body, bs, num_blk),
        grid=(num_blk,),
        in_specs=[
            pl.BlockSpec(memory_space=pl.ANY),
            pl.BlockSpec((1, bs, bs), lambda d: (d, 0, 0)),
            pl.BlockSpec(memory_space=pl.ANY),
            pl.BlockSpec(memory_space=pltpu.SMEM),
            pl.BlockSpec(memory_space=pltpu.SMEM),
        ],
        out_specs=pl.BlockSpec(memory_space=pl.ANY),
        out_shape=jax.ShapeDtypeStruct((n, n), jnp.float32),
        scratch_shapes=[
            pltpu.VMEM((2, bs, bs), jnp.float32),
            pltpu.VMEM((2, bs, bs), jnp.float32),
            pltpu.VMEM((2, bs, bs), jnp.float32),
            pltpu.SemaphoreType.DMA((2, 3)),
        ],
        input_output_aliases={0: 0},
        compiler_params=pltpu.CompilerParams(
            dimension_semantics=("arbitrary",),
        ),
    )(tc_out, sc_bits, edge_probs, bi_sc, bj_sc)


# speedup vs baseline: 1.1870x; 1.1870x over previous
import functools

import jax
import jax.numpy as jnp
import numpy as np
from jax import lax
from jax.experimental import pallas as pl
from jax.experimental.pallas import tpu as pltpu
from jax.experimental.pallas import tpu_sc as plsc

_BS = 1024


def _threefry_bits_u32(x1):
    ks0 = jnp.uint32(0)
    ks1 = jnp.uint32(42)
    ks2 = jnp.uint32(0x1BD11BDA) ^ ks0 ^ ks1
    x0 = jnp.zeros_like(x1) + ks0

    def rounds(x0, x1, rots):
        for d in rots:
            x0 = x0 + x1
            x1 = (x1 << d) | (x1 >> (32 - d))
            x1 = x1 ^ x0
        return x0, x1

    r_a = (13, 15, 26, 6)
    r_b = (17, 29, 16, 24)
    for i, (a0, a1, rots) in enumerate(
        [(ks1, ks2, r_a), (ks2, ks0, r_b), (ks0, ks1, r_a),
         (ks1, ks2, r_b), (ks2, ks0, r_a)]
    ):
        x0, x1 = rounds(x0, x1, rots)
        x0 = x0 + a0
        x1 = x1 + a1 + jnp.uint32(i + 1)
    return x0 ^ x1


def _body(n, bs, num_pairs, bi_ref, bj_ref, probs_ref, out_ref,
          up_buf, tr_buf, iota_buf, sems):
    p = pl.program_id(0)
    slot = lax.rem(p, 2)
    bi = bi_ref[p]
    bj = bj_ref[p]

    @pl.when(p == 0)
    def _init_iota():
        rl = lax.broadcasted_iota(jnp.uint32, (bs, bs), 0)
        cl = lax.broadcasted_iota(jnp.uint32, (bs, bs), 1)
        iota_buf[...] = rl * jnp.uint32(n) + cl

    def copy_desc(buf, row_blk, col_blk, sem_idx):
        return pltpu.make_async_copy(
            buf.at[slot],
            out_ref.at[pl.ds(row_blk * bs, bs), pl.ds(col_blk * bs, bs)],
            sems.at[slot, sem_idx],
        )

    @pl.when(p >= 2)
    def _drain_prev():
        copy_desc(up_buf, bi, bj, 0).wait()
        copy_desc(tr_buf, bj, bi, 1).wait()

    base = (bi * (bs * n) + bj * bs + 42).astype(jnp.uint32)
    bits = _threefry_bits_u32(iota_buf[...] + base)
    fbits = (bits >> 9) | jnp.uint32(0x3F800000)
    noise = lax.bitcast_convert_type(fbits, jnp.float32) - 1.0
    bern = (noise < probs_ref[...]).astype(jnp.float32)
    bern_t = bern.T
    diag = bi == bj

    @pl.when(jnp.logical_not(diag))
    def _off_diag():
        up_buf[slot] = bern
        tr_buf[slot] = bern_t
        copy_desc(tr_buf, bj, bi, 1).start()

    @pl.when(diag)
    def _diag():
        rl = lax.broadcasted_iota(jnp.int32, (bs, bs), 0)
        cl = lax.broadcasted_iota(jnp.int32, (bs, bs), 1)
        up_buf[slot] = jnp.where(rl > cl, bern_t, bern)
        copy_desc(up_buf, bj, bi, 1).start()

    copy_desc(up_buf, bi, bj, 0).start()

    @pl.when(p == num_pairs - 1)
    def _drain_tail():
        copy_desc(up_buf, bi, bj, 0).wait()
        copy_desc(tr_buf, bj, bi, 1).wait()
        if num_pairs >= 2:
            other = 1 - slot
            pltpu.make_async_copy(
                up_buf.at[other],
                out_ref.at[pl.ds(0, bs), pl.ds(0, bs)],
                sems.at[other, 0],
            ).wait()
            pltpu.make_async_copy(
                tr_buf.at[other],
                out_ref.at[pl.ds(0, bs), pl.ds(0, bs)],
                sems.at[other, 1],
            ).wait()


_SC_BLOCKS = [(d, d) for d in range(8)] + [(0, 1)]


def _sc_bits_block(n, bs):
    mesh = plsc.VectorSubcoreMesh(core_axis_name="c", subcore_axis_name="s")
    rows_per_w = bs // 32
    nblk = len(_SC_BLOCKS)

    @functools.partial(
        pl.kernel,
        mesh=mesh,
        out_type=jax.ShapeDtypeStruct((nblk, bs, bs), jnp.uint32),
        scratch_types=[
            pltpu.VMEM((16,), jnp.uint32),
            pltpu.VMEM((2, bs), jnp.uint32),
            pltpu.SemaphoreType.DMA((2,)),
        ],
    )
    def k(lane_hbm, out_hbm, lane_v, row_v, sems):
        wid = lax.axis_index("c") * 16 + lax.axis_index("s")
        pltpu.sync_copy(lane_hbm, lane_v)
        lane = lane_v[...]

        for b, (bi, bj) in enumerate(_SC_BLOCKS):
            base_const = jnp.uint32((bi * bs * n + bj * bs + 42) & 0xFFFFFFFF)

            @pl.loop(0, rows_per_w)
            def _row(i, b=b, base_const=base_const):
                row = wid * rows_per_w + i
                slot = lax.rem(i, 2)
                rbase = row.astype(jnp.uint32) * jnp.uint32(n) + base_const

                @pl.when(i >= 2)
                def _():
                    pltpu.make_async_copy(
                        row_v.at[slot], out_hbm.at[b, row], sems.at[slot]
                    ).wait()

                @pl.loop(0, bs, step=32)
                def _chunk(c, rbase=rbase, slot=slot):
                    x1a = lane + (rbase + c.astype(jnp.uint32))
                    row_v[slot, pl.ds(c, 16)] = _threefry_bits_u32(x1a)
                    row_v[slot, pl.ds(c + 16, 16)] = _threefry_bits_u32(
                        x1a + jnp.uint32(16))

                pltpu.make_async_copy(
                    row_v.at[slot], out_hbm.at[b, row], sems.at[slot]
                ).start()

            pltpu.make_async_copy(
                row_v.at[lax.rem(rows_per_w - 1, 2)],
                out_hbm.at[b, 0], sems.at[lax.rem(rows_per_w - 1, 2)],
            ).wait()
            pltpu.make_async_copy(
                row_v.at[lax.rem(rows_per_w, 2)],
                out_hbm.at[b, 0], sems.at[lax.rem(rows_per_w, 2)],
            ).wait()

    lane_arr = jnp.arange(16, dtype=jnp.uint32)
    return k(lane_arr)


def _stitch_body(bs, num_blk, tc_in_ref, bits_ref, probs_hbm, bi_ref, bj_ref,
                 out_ref, pbuf, up_buf, tr_buf, sems):
    del tc_in_ref
    d = pl.program_id(0)
    slot = lax.rem(d, 2)
    bi = bi_ref[d]
    bj = bj_ref[d]

    def copy_desc(buf, row_blk, col_blk, sem_idx):
        return pltpu.make_async_copy(
            buf.at[slot],
            out_ref.at[pl.ds(row_blk * bs, bs), pl.ds(col_blk * bs, bs)],
            sems.at[slot, sem_idx],
        )

    probs_in = pltpu.make_async_copy(
        probs_hbm.at[pl.ds(bi * bs, bs), pl.ds(bj * bs, bs)],
        pbuf.at[slot],
        sems.at[slot, 2],
    )
    probs_in.start()

    @pl.when(d >= 2)
    def _drain_prev():
        copy_desc(up_buf, bi, bj, 0).wait()
        copy_desc(tr_buf, bj, bi, 1).wait()

    probs_in.wait()

    fbits = (bits_ref[0] >> 9) | jnp.uint32(0x3F800000)
    noise = lax.bitcast_convert_type(fbits, jnp.float32) - 1.0
    bern = (noise < pbuf[slot]).astype(jnp.float32)
    bern_t = bern.T
    diag = bi == bj

    @pl.when(jnp.logical_not(diag))
    def _off_diag():
        up_buf[slot] = bern
        tr_buf[slot] = bern_t
        copy_desc(tr_buf, bj, bi, 1).start()

    @pl.when(diag)
    def _diag():
        rl = lax.broadcasted_iota(jnp.int32, (bs, bs), 0)
        cl = lax.broadcasted_iota(jnp.int32, (bs, bs), 1)
        up_buf[slot] = jnp.where(rl > cl, bern_t, bern)
        copy_desc(up_buf, bj, bi, 1).start()

    copy_desc(up_buf, bi, bj, 0).start()

    @pl.when(d == num_blk - 1)
    def _drain_tail():
        copy_desc(up_buf, bi, bj, 0).wait()
        copy_desc(tr_buf, bj, bi, 1).wait()
        if num_blk >= 2:
            other = 1 - slot
            pltpu.make_async_copy(
                up_buf.at[other],
                out_ref.at[pl.ds(0, bs), pl.ds(0, bs)],
                sems.at[other, 0],
            ).wait()
            pltpu.make_async_copy(
                tr_buf.at[other],
                out_ref.at[pl.ds(0, bs), pl.ds(0, bs)],
                sems.at[other, 1],
            ).wait()


def kernel(edge_probs):
    n = edge_probs.shape[0]
    bs = _BS
    nb = n // bs
    sc_set = set(_SC_BLOCKS)
    pairs = [(i, j) for i in range(nb) for j in range(i, nb)
             if (i, j) not in sc_set]
    bi_arr = jnp.asarray(np.array([ij[0] for ij in pairs], dtype=np.int32))
    bj_arr = jnp.asarray(np.array([ij[1] for ij in pairs], dtype=np.int32))
    num_pairs = len(pairs)

    grid_spec = pltpu.PrefetchScalarGridSpec(
        num_scalar_prefetch=2,
        grid=(num_pairs,),
        in_specs=[
            pl.BlockSpec((bs, bs), lambda p, bi, bj: (bi[p], bj[p])),
        ],
        out_specs=pl.BlockSpec(memory_space=pl.ANY),
        scratch_shapes=[
            pltpu.VMEM((2, bs, bs), jnp.float32),
            pltpu.VMEM((2, bs, bs), jnp.float32),
            pltpu.VMEM((bs, bs), jnp.uint32),
            pltpu.SemaphoreType.DMA((2, 2)),
        ],
    )
    tc_out = pl.pallas_call(
        functools.partial(_body, n, bs, num_pairs),
        grid_spec=grid_spec,
        out_shape=jax.ShapeDtypeStruct((n, n), jnp.float32),
        compiler_params=pltpu.CompilerParams(
            dimension_semantics=("arbitrary",),
        ),
    )(bi_arr, bj_arr, edge_probs)

    sc_bits = _sc_bits_block(n, bs)
    num_blk = len(_SC_BLOCKS)
    bi_sc = jnp.asarray([b[0] for b in _SC_BLOCKS], dtype=jnp.int32)
    bj_sc = jnp.asarray([b[1] for b in _SC_BLOCKS], dtype=jnp.int32)
    return pl.pallas_call(
        functools.partial(_stitch_body, bs, num_blk),
        grid=(num_blk,),
        in_specs=[
            pl.BlockSpec(memory_space=pl.ANY),
            pl.BlockSpec((1, bs, bs), lambda d: (d, 0, 0)),
            pl.BlockSpec(memory_space=pl.ANY),
            pl.BlockSpec(memory_space=pltpu.SMEM),
            pl.BlockSpec(memory_space=pltpu.SMEM),
        ],
        out_specs=pl.BlockSpec(memory_space=pl.ANY),
        out_shape=jax.ShapeDtypeStruct((n, n), jnp.float32),
        scratch_shapes=[
            pltpu.VMEM((2, bs, bs), jnp.float32),
            pltpu.VMEM((2, bs, bs), jnp.float32),
            pltpu.VMEM((2, bs, bs), jnp.float32),
            pltpu.SemaphoreType.DMA((2, 3)),
        ],
        input_output_aliases={0: 0},
        compiler_params=pltpu.CompilerParams(
            dimension_semantics=("arbitrary",),
        ),
    )(tc_out, sc_bits, edge_probs, bi_sc, bj_sc)


# speedup vs baseline: 1.2345x; 1.0400x over previous
import functools

import jax
import jax.numpy as jnp
import numpy as np
from jax import lax
from jax.experimental import pallas as pl
from jax.experimental.pallas import tpu as pltpu
from jax.experimental.pallas import tpu_sc as plsc

_BS = 1024


def _threefry_bits_u32(x1):
    ks0 = jnp.uint32(0)
    ks1 = jnp.uint32(42)
    ks2 = jnp.uint32(0x1BD11BDA) ^ ks0 ^ ks1
    x0 = jnp.zeros_like(x1) + ks0

    def rounds(x0, x1, rots):
        for d in rots:
            x0 = x0 + x1
            x1 = (x1 << d) | (x1 >> (32 - d))
            x1 = x1 ^ x0
        return x0, x1

    r_a = (13, 15, 26, 6)
    r_b = (17, 29, 16, 24)
    for i, (a0, a1, rots) in enumerate(
        [(ks1, ks2, r_a), (ks2, ks0, r_b), (ks0, ks1, r_a),
         (ks1, ks2, r_b), (ks2, ks0, r_a)]
    ):
        x0, x1 = rounds(x0, x1, rots)
        x0 = x0 + a0
        x1 = x1 + a1 + jnp.uint32(i + 1)
    return x0 ^ x1


def _body(n, bs, num_pairs, bi_ref, bj_ref, probs_ref, out_ref,
          up_buf, tr_buf, iota_buf, sems):
    p = pl.program_id(0)
    slot = lax.rem(p, 2)
    bi = bi_ref[p]
    bj = bj_ref[p]

    @pl.when(p == 0)
    def _init_iota():
        rl = lax.broadcasted_iota(jnp.uint32, (bs, bs), 0)
        cl = lax.broadcasted_iota(jnp.uint32, (bs, bs), 1)
        iota_buf[...] = rl * jnp.uint32(n) + cl

    def copy_desc(buf, row_blk, col_blk, sem_idx):
        return pltpu.make_async_copy(
            buf.at[slot],
            out_ref.at[pl.ds(row_blk * bs, bs), pl.ds(col_blk * bs, bs)],
            sems.at[slot, sem_idx],
        )

    @pl.when(p >= 2)
    def _drain_prev():
        copy_desc(up_buf, bi, bj, 0).wait()
        copy_desc(tr_buf, bj, bi, 1).wait()

    base = (bi * (bs * n) + bj * bs + 42).astype(jnp.uint32)
    bits = _threefry_bits_u32(iota_buf[...] + base)
    fbits = (bits >> 9) | jnp.uint32(0x3F800000)
    noise = lax.bitcast_convert_type(fbits, jnp.float32) - 1.0
    bern = (noise < probs_ref[...]).astype(jnp.float32)
    bern_t = bern.T
    diag = bi == bj

    @pl.when(jnp.logical_not(diag))
    def _off_diag():
        up_buf[slot] = bern
        tr_buf[slot] = bern_t
        copy_desc(tr_buf, bj, bi, 1).start()

    @pl.when(diag)
    def _diag():
        rl = lax.broadcasted_iota(jnp.int32, (bs, bs), 0)
        cl = lax.broadcasted_iota(jnp.int32, (bs, bs), 1)
        up_buf[slot] = jnp.where(rl > cl, bern_t, bern)
        copy_desc(up_buf, bj, bi, 1).start()

    copy_desc(up_buf, bi, bj, 0).start()

    @pl.when(p == num_pairs - 1)
    def _drain_tail():
        copy_desc(up_buf, bi, bj, 0).wait()
        copy_desc(tr_buf, bj, bi, 1).wait()
        if num_pairs >= 2:
            other = 1 - slot
            pltpu.make_async_copy(
                up_buf.at[other],
                out_ref.at[pl.ds(0, bs), pl.ds(0, bs)],
                sems.at[other, 0],
            ).wait()
            pltpu.make_async_copy(
                tr_buf.at[other],
                out_ref.at[pl.ds(0, bs), pl.ds(0, bs)],
                sems.at[other, 1],
            ).wait()


_SC_DIAG = 8


def _sc_bits_block(n, bs):
    mesh = plsc.VectorSubcoreMesh(core_axis_name="c", subcore_axis_name="s")
    rows_per_w = bs // 32

    @functools.partial(
        pl.kernel,
        mesh=mesh,
        out_type=jax.ShapeDtypeStruct((_SC_DIAG, bs, bs), jnp.uint32),
        scratch_types=[
            pltpu.VMEM((16,), jnp.uint32),
            pltpu.VMEM((2, bs), jnp.uint32),
            pltpu.SemaphoreType.DMA((2,)),
        ],
    )
    def k(lane_hbm, out_hbm, lane_v, row_v, sems):
        wid = lax.axis_index("c") * 16 + lax.axis_index("s")
        pltpu.sync_copy(lane_hbm, lane_v)
        lane = lane_v[...]

        @pl.loop(0, _SC_DIAG)
        def _blk(b):
            @pl.loop(0, rows_per_w)
            def _row(i):
                row = wid * rows_per_w + i
                slot = lax.rem(i, 2)
                rbase = ((b * bs + row) * n + b * bs + 42).astype(jnp.uint32)

                @pl.when(i >= 2)
                def _():
                    pltpu.make_async_copy(
                        row_v.at[slot], out_hbm.at[b, row], sems.at[slot]
                    ).wait()

                @pl.loop(0, bs, step=32)
                def _chunk(c):
                    x1a = lane + (rbase + c.astype(jnp.uint32))
                    row_v[slot, pl.ds(c, 16)] = _threefry_bits_u32(x1a)
                    row_v[slot, pl.ds(c + 16, 16)] = _threefry_bits_u32(
                        x1a + jnp.uint32(16))

                pltpu.make_async_copy(
                    row_v.at[slot], out_hbm.at[b, row], sems.at[slot]
                ).start()

            pltpu.make_async_copy(
                row_v.at[lax.rem(rows_per_w - 1, 2)],
                out_hbm.at[b, 0], sems.at[lax.rem(rows_per_w - 1, 2)],
            ).wait()
            pltpu.make_async_copy(
                row_v.at[lax.rem(rows_per_w, 2)],
                out_hbm.at[b, 0], sems.at[lax.rem(rows_per_w, 2)],
            ).wait()

    lane_arr = jnp.arange(16, dtype=jnp.uint32)
    return k(lane_arr)


def _diag_finish_body(bs, tc_in_ref, bits_ref, probs_ref, out_ref, vbuf, sems):
    del tc_in_ref
    d = pl.program_id(0)
    slot = lax.rem(d, 2)

    def copy_desc():
        return pltpu.make_async_copy(
            vbuf.at[slot],
            out_ref.at[pl.ds(d * bs, bs), pl.ds(d * bs, bs)],
            sems.at[slot],
        )

    @pl.when(d >= 2)
    def _drain_prev():
        copy_desc().wait()

    fbits = (bits_ref[0] >> 9) | jnp.uint32(0x3F800000)
    noise = lax.bitcast_convert_type(fbits, jnp.float32) - 1.0
    bern = (noise < probs_ref[...]).astype(jnp.float32)
    rl = lax.broadcasted_iota(jnp.int32, (bs, bs), 0)
    cl = lax.broadcasted_iota(jnp.int32, (bs, bs), 1)
    vbuf[slot] = jnp.where(rl > cl, bern.T, bern)
    copy_desc().start()

    @pl.when(d == _SC_DIAG - 1)
    def _drain_tail():
        copy_desc().wait()
        if _SC_DIAG >= 2:
            other = 1 - slot
            pltpu.make_async_copy(
                vbuf.at[other],
                out_ref.at[pl.ds(0, bs), pl.ds(0, bs)],
                sems.at[other],
            ).wait()


def kernel(edge_probs):
    n = edge_probs.shape[0]
    bs = _BS
    nb = n // bs
    sc_diag = set((d, d) for d in range(_SC_DIAG))
    pairs = [(i, j) for i in range(nb) for j in range(i, nb)
             if (i, j) not in sc_diag]
    bi_arr = jnp.asarray(np.array([ij[0] for ij in pairs], dtype=np.int32))
    bj_arr = jnp.asarray(np.array([ij[1] for ij in pairs], dtype=np.int32))
    num_pairs = len(pairs)

    grid_spec = pltpu.PrefetchScalarGridSpec(
        num_scalar_prefetch=2,
        grid=(num_pairs,),
        in_specs=[
            pl.BlockSpec((bs, bs), lambda p, bi, bj: (bi[p], bj[p])),
        ],
        out_specs=pl.BlockSpec(memory_space=pl.ANY),
        scratch_shapes=[
            pltpu.VMEM((2, bs, bs), jnp.float32),
            pltpu.VMEM((2, bs, bs), jnp.float32),
            pltpu.VMEM((bs, bs), jnp.uint32),
            pltpu.SemaphoreType.DMA((2, 2)),
        ],
    )
    tc_out = pl.pallas_call(
        functools.partial(_body, n, bs, num_pairs),
        grid_spec=grid_spec,
        out_shape=jax.ShapeDtypeStruct((n, n), jnp.float32),
        compiler_params=pltpu.CompilerParams(
            dimension_semantics=("arbitrary",),
        ),
    )(bi_arr, bj_arr, edge_probs)

    sc_bits = _sc_bits_block(n, bs)
    return pl.pallas_call(
        functools.partial(_diag_finish_body, bs),
        grid=(_SC_DIAG,),
        in_specs=[
            pl.BlockSpec(memory_space=pl.ANY),
            pl.BlockSpec((1, bs, bs), lambda d: (d, 0, 0)),
            pl.BlockSpec((bs, bs), lambda d: (d, d)),
        ],
        out_specs=pl.BlockSpec(memory_space=pl.ANY),
        out_shape=jax.ShapeDtypeStruct((n, n), jnp.float32),
        scratch_shapes=[
            pltpu.VMEM((2, bs, bs), jnp.float32),
            pltpu.SemaphoreType.DMA((2,)),
        ],
        input_output_aliases={0: 0},
        compiler_params=pltpu.CompilerParams(
            dimension_semantics=("arbitrary",),
        ),
    )(tc_out, sc_bits, edge_probs)
